# Initial kernel scaffold; baseline (speedup 1.0000x reference)
import jax
import jax.numpy as jnp
from jax.experimental import pallas as pl


def kernel(x, edge_index, W1, b1, W2, b2, W3, b3):
    raise NotImplementedError("write your pallas kernel here")


# speedup vs baseline: 18.9635x; 18.9635x over previous
import functools

import jax
import jax.numpy as jnp
from jax import lax
from jax.experimental import pallas as pl
from jax.experimental.pallas import tpu as pltpu
from jax.experimental.pallas import tpu_sc as plsc

N = 10000
NP = 10240
E = 320000
H = 16
D_OUT = 100
NC = 2
NS = 16
NW = NC * NS
EPW_REAL = E // NW
CH = 128
NCHUNK = 80
EPW = NCHUNK * CH
RPT = NP // NS

_mesh_cache = []


def _mesh():
    if not _mesh_cache:
        _mesh_cache.append(plsc.VectorSubcoreMesh(
            core_axis_name="c", subcore_axis_name="s", num_cores=NC, num_subcores=NS
        ))
    return _mesh_cache[0]


def _fill(ref, value):
    v = jnp.full((H,), value, jnp.float32)
    for i in range(ref.shape[0]):
        ref[i, :] = v


def _sc_degree_body(src_hbm, dst_hbm, out_hbm, idx_s, idx_d, buf, acc_s, acc_d):
    c = lax.axis_index("c")
    s = lax.axis_index("s")
    wid = s * NC + c
    base = s * RPT

    _fill(buf, 0.0)
    for k in range(RPT // CH):
        pltpu.sync_copy(buf, acc_s.at[pl.ds(base + k * CH, CH)])
        pltpu.sync_copy(buf, acc_d.at[pl.ds(base + k * CH, CH)])
    pltpu.sync_copy(src_hbm.at[wid], idx_s)
    pltpu.sync_copy(dst_hbm.at[wid], idx_d)
    _fill(buf, 1.0)
    plsc.subcore_barrier()

    for j in range(NCHUNK):
        pltpu.sync_copy(buf, acc_s.at[idx_s.at[j]], add=True)
        pltpu.sync_copy(buf, acc_d.at[idx_d.at[j]], add=True)
    plsc.subcore_barrier()
    pltpu.sync_copy(acc_s.at[pl.ds(base, RPT)],
                    out_hbm.at[pl.ds(c * (2 * NP) + base, RPT)])
    pltpu.sync_copy(acc_d.at[pl.ds(base, RPT)],
                    out_hbm.at[pl.ds(c * (2 * NP) + NP + base, RPT)])


def _sc_segsum_body(t_hbm, src_hbm, dst_hbm, out_hbm, idx_s, idx_d, rows, t_sp, acc, sem):
    c = lax.axis_index("c")
    s = lax.axis_index("s")
    wid = s * NC + c
    base = s * RPT

    _fill(rows, 0.0)
    for k in range(RPT // CH):
        pltpu.sync_copy(rows, acc.at[pl.ds(base + k * CH, CH)])
    pltpu.sync_copy(t_hbm.at[pl.ds(base, RPT)], t_sp.at[pl.ds(base, RPT)])
    pltpu.sync_copy(src_hbm.at[wid], idx_s)
    pltpu.sync_copy(dst_hbm.at[wid], idx_d)
    plsc.subcore_barrier()

    for j in range(NCHUNK):
        pltpu.async_copy(t_sp.at[idx_s.at[j]], rows, sem).wait()
        pltpu.sync_copy(rows, acc.at[idx_d.at[j]], add=True)
    plsc.subcore_barrier()
    pltpu.sync_copy(acc.at[pl.ds(base, RPT)],
                    out_hbm.at[pl.ds(c * NP + base, RPT)])


def _make_sc_degree(interpret=False):
    return functools.partial(
        pl.kernel,
        out_type=jax.ShapeDtypeStruct((NC * 2 * NP, H), jnp.float32),
        mesh=_mesh(),
        scratch_types=[
            pltpu.VMEM((NCHUNK, CH), jnp.int32),
            pltpu.VMEM((NCHUNK, CH), jnp.int32),
            pltpu.VMEM((CH, H), jnp.float32),
            pltpu.VMEM_SHARED((NP, H), jnp.float32),
            pltpu.VMEM_SHARED((NP, H), jnp.float32),
        ],
        compiler_params=pltpu.CompilerParams(use_tc_tiling_on_sc=False),
        interpret=interpret,
    )(_sc_degree_body)


def _make_sc_segsum(interpret=False):
    return functools.partial(
        pl.kernel,
        out_type=jax.ShapeDtypeStruct((NC * NP, H), jnp.float32),
        mesh=_mesh(),
        scratch_types=[
            pltpu.VMEM((NCHUNK, CH), jnp.int32),
            pltpu.VMEM((NCHUNK, CH), jnp.int32),
            pltpu.VMEM((CH, H), jnp.float32),
            pltpu.VMEM_SHARED((NP, H), jnp.float32),
            pltpu.VMEM_SHARED((NP, H), jnp.float32),
            pltpu.SemaphoreType.DMA,
        ],
        compiler_params=pltpu.CompilerParams(use_tc_tiling_on_sc=False),
        interpret=interpret,
    )(_sc_segsum_body)


_sc_cache = {}


def _sc_degree(*args):
    if "deg" not in _sc_cache:
        _sc_cache["deg"] = _make_sc_degree()
    return _sc_cache["deg"](*args)


def _sc_segsum(*args):
    if "seg" not in _sc_cache:
        _sc_cache["seg"] = _make_sc_segsum()
    return _sc_cache["seg"](*args)


def _tc0_body(deg_ref, x_ref, w1_ref, t1_ref, so_ref, si_ref):
    row = lax.broadcasted_iota(jnp.int32, (NP, H), 0)
    valid = row < N
    deg_out = jnp.maximum(deg_ref[0:NP, :] + deg_ref[2 * NP:3 * NP, :], 1.0)
    deg_in = jnp.maximum(deg_ref[NP:2 * NP, :] + deg_ref[3 * NP:4 * NP, :], 1.0)
    so = jnp.where(valid, lax.rsqrt(deg_out), 0.0)
    si = jnp.where(valid, lax.rsqrt(deg_in), 0.0)
    so_ref[...] = so
    si_ref[...] = si
    t1 = jnp.dot(x_ref[...], w1_ref[...], preferred_element_type=jnp.float32)
    t1_ref[...] = t1 * so


_tc0 = pl.pallas_call(
    _tc0_body,
    out_shape=(
        jax.ShapeDtypeStruct((NP, H), jnp.float32),
        jax.ShapeDtypeStruct((NP, H), jnp.float32),
        jax.ShapeDtypeStruct((NP, H), jnp.float32),
    ),
)


def _tc1_body(p_ref, si_ref, so_ref, b_ref, w_ref, out_ref):
    agg = (p_ref[0:NP, :] + p_ref[NP:2 * NP, :]) * si_ref[...]
    h = jnp.maximum(agg + b_ref[...], 0.0)
    out_ref[...] = jnp.dot(h, w_ref[...], preferred_element_type=jnp.float32) * so_ref[...]


_tc1 = pl.pallas_call(
    _tc1_body,
    out_shape=jax.ShapeDtypeStruct((NP, H), jnp.float32),
)


def _tc2_body(p_ref, si_ref, so_ref, b_ref, out_ref):
    agg = (p_ref[0:NP, :] + p_ref[NP:2 * NP, :]) * si_ref[...]
    out_ref[...] = jnp.maximum(agg + b_ref[...], 0.0) * so_ref[...]


_tc2 = pl.pallas_call(
    _tc2_body,
    out_shape=jax.ShapeDtypeStruct((NP, H), jnp.float32),
)


def _tc3_body(p_ref, si_ref, b_ref, w_ref, out_ref):
    agg = (p_ref[0:NP, :] + p_ref[NP:2 * NP, :]) * si_ref[...]
    out_ref[...] = jnp.dot(agg, w_ref[...], preferred_element_type=jnp.float32) + b_ref[...]


_tc3 = pl.pallas_call(
    _tc3_body,
    out_shape=jax.ShapeDtypeStruct((NP, D_OUT), jnp.float32),
)


def kernel(x, edge_index, W1, b1, W2, b2, W3, b3):
    src = edge_index[0].astype(jnp.int32).reshape(NW, EPW_REAL)
    dst = edge_index[1].astype(jnp.int32).reshape(NW, EPW_REAL)
    padw = jnp.full((NW, EPW - EPW_REAL), N, jnp.int32)
    src_r = jnp.concatenate([src, padw], axis=1).reshape(NW, NCHUNK, CH)
    dst_r = jnp.concatenate([dst, padw], axis=1).reshape(NW, NCHUNK, CH)
    x_p = jnp.pad(x, ((0, NP - N), (0, 0)))

    deg = _sc_degree(src_r, dst_r)
    t1, so, si = _tc0(deg, x_p, W1)
    p1 = _sc_segsum(t1, src_r, dst_r)
    t2 = _tc1(p1, si, so, b1.reshape(1, H), W2)
    p2 = _sc_segsum(t2, src_r, dst_r)
    t3 = _tc2(p2, si, so, b2.reshape(1, H))
    p3 = _sc_segsum(t3, src_r, dst_r)
    out = _tc3(p3, si, b3.reshape(1, D_OUT), W3)
    return out[:N]


# speedup vs baseline: 24.2714x; 1.2799x over previous
import functools

import jax
import jax.numpy as jnp
from jax import lax
from jax.experimental import pallas as pl
from jax.experimental.pallas import tpu as pltpu
from jax.experimental.pallas import tpu_sc as plsc

N = 10000
NP = 10240
E = 320000
H = 16
D_OUT = 100
NC = 2
NS = 16
NW = NC * NS
EPW_REAL = E // NW
CH = 2048
NCHUNK = 5
EPW = NCHUNK * CH
RPT = NP // NS

_CPARAMS = None


def _cparams():
    return pltpu.CompilerParams(use_tc_tiling_on_sc=False)


_mesh_cache = []


def _mesh():
    if not _mesh_cache:
        _mesh_cache.append(plsc.VectorSubcoreMesh(
            core_axis_name="c", subcore_axis_name="s", num_cores=NC, num_subcores=NS
        ))
    return _mesh_cache[0]


def _sc_degree_body(src_hbm, dst_hbm, out_hbm, idx_s, idx_d, ones_v, acc_s, acc_d, sem):
    c = lax.axis_index("c")
    s = lax.axis_index("s")
    wid = s * NC + c
    base = s * RPT

    z = jnp.zeros((H,), jnp.float32)
    for i in range(RPT // H):
        ones_v[pl.ds(i * H, H)] = z
    pltpu.sync_copy(ones_v.at[pl.ds(0, RPT)], acc_s.at[pl.ds(base, RPT)])
    pltpu.sync_copy(ones_v.at[pl.ds(0, RPT)], acc_d.at[pl.ds(base, RPT)])
    pltpu.sync_copy(src_hbm.at[pl.ds(wid * EPW, EPW)], idx_s)
    pltpu.sync_copy(dst_hbm.at[pl.ds(wid * EPW, EPW)], idx_d)
    o = jnp.ones((H,), jnp.float32)
    for i in range(CH // H):
        ones_v[pl.ds(i * H, H)] = o
    plsc.subcore_barrier()

    descs = []
    for j in range(NCHUNK):
        descs.append(pltpu.async_copy(
            ones_v, acc_s.at[idx_s.at[pl.ds(j * CH, CH)]], sem, add=True))
        descs.append(pltpu.async_copy(
            ones_v, acc_d.at[idx_d.at[pl.ds(j * CH, CH)]], sem, add=True))
    for d in descs:
        d.wait()
    plsc.subcore_barrier()
    pltpu.sync_copy(acc_s.at[pl.ds(base, RPT)],
                    out_hbm.at[pl.ds(c * (2 * NP) + base, RPT)])
    pltpu.sync_copy(acc_d.at[pl.ds(base, RPT)],
                    out_hbm.at[pl.ds(c * (2 * NP) + NP + base, RPT)])


def _make_sc_degree(interpret=False):
    return functools.partial(
        pl.kernel,
        out_type=jax.ShapeDtypeStruct((NC * 2 * NP,), jnp.float32),
        mesh=_mesh(),
        scratch_types=[
            pltpu.VMEM((EPW,), jnp.int32),
            pltpu.VMEM((EPW,), jnp.int32),
            pltpu.VMEM((CH,), jnp.float32),
            pltpu.VMEM_SHARED((NP,), jnp.float32),
            pltpu.VMEM_SHARED((NP,), jnp.float32),
            pltpu.SemaphoreType.DMA,
        ],
        compiler_params=_cparams(),
        interpret=interpret,
    )(_sc_degree_body)


def _sc_segsum_body(t_hbm, src_hbm, dst_hbm, out_hbm,
                    idx_s, idx_d, rows0, rows1, t_sp, acc, sem_g, sem_s):
    c = lax.axis_index("c")
    s = lax.axis_index("s")
    wid = s * NC + c
    base = s * RPT
    rows = (rows0, rows1)

    z = jnp.zeros((H,), jnp.float32)
    for i in range(RPT):
        rows0[i, :] = z
    pltpu.sync_copy(rows0.at[pl.ds(0, RPT)], acc.at[pl.ds(base, RPT)])
    pltpu.sync_copy(t_hbm.at[pl.ds(base, RPT)], t_sp.at[pl.ds(base, RPT)])
    pltpu.sync_copy(src_hbm.at[pl.ds(wid * EPW, EPW)], idx_s)
    pltpu.sync_copy(dst_hbm.at[pl.ds(wid * EPW, EPW)], idx_d)
    plsc.subcore_barrier()

    def gather(j, buf):
        return pltpu.async_copy(
            t_sp.at[idx_s.at[pl.ds(j * CH, CH)]], buf, sem_g)

    def scatter(j, buf):
        return pltpu.async_copy(
            buf, acc.at[idx_d.at[pl.ds(j * CH, CH)]], sem_s, add=True)

    g = [None] * NCHUNK
    sc = [None] * NCHUNK
    g[0] = gather(0, rows[0])
    for j in range(NCHUNK):
        g[j].wait()
        if j + 1 < NCHUNK:
            if j >= 1:
                sc[j - 1].wait()
            g[j + 1] = gather(j + 1, rows[(j + 1) % 2])
        sc[j] = scatter(j, rows[j % 2])
    sc[NCHUNK - 2].wait()
    sc[NCHUNK - 1].wait()
    plsc.subcore_barrier()
    pltpu.sync_copy(acc.at[pl.ds(base, RPT)],
                    out_hbm.at[pl.ds(c * NP + base, RPT)])


def _make_sc_segsum(interpret=False):
    return functools.partial(
        pl.kernel,
        out_type=jax.ShapeDtypeStruct((NC * NP, H), jnp.float32),
        mesh=_mesh(),
        scratch_types=[
            pltpu.VMEM((EPW,), jnp.int32),
            pltpu.VMEM((EPW,), jnp.int32),
            pltpu.VMEM((CH, H), jnp.float32),
            pltpu.VMEM((CH, H), jnp.float32),
            pltpu.VMEM_SHARED((NP, H), jnp.float32),
            pltpu.VMEM_SHARED((NP, H), jnp.float32),
            pltpu.SemaphoreType.DMA,
            pltpu.SemaphoreType.DMA,
        ],
        compiler_params=_cparams(),
        interpret=interpret,
    )(_sc_segsum_body)


_sc_cache = {}


def _sc_degree(*args):
    if "deg" not in _sc_cache:
        _sc_cache["deg"] = _make_sc_degree()
    return _sc_cache["deg"](*args)


def _sc_segsum(*args):
    if "seg" not in _sc_cache:
        _sc_cache["seg"] = _make_sc_segsum()
    return _sc_cache["seg"](*args)


def _tcA_body(deg_ref, so_ref, si_ref):
    B = NP // 128
    d = deg_ref[...]
    deg_out = jnp.maximum(d[0:B, :] + d[2 * B:3 * B, :], 1.0)
    deg_in = jnp.maximum(d[B:2 * B, :] + d[3 * B:4 * B, :], 1.0)
    node = (lax.broadcasted_iota(jnp.int32, (B, 128), 0) * 128
            + lax.broadcasted_iota(jnp.int32, (B, 128), 1))
    valid = node < N
    so_ref[...] = jnp.where(valid, lax.rsqrt(deg_out), 0.0)
    si_ref[...] = jnp.where(valid, lax.rsqrt(deg_in), 0.0)


_tcA = pl.pallas_call(
    _tcA_body,
    out_shape=(
        jax.ShapeDtypeStruct((NP // 128, 128), jnp.float32),
        jax.ShapeDtypeStruct((NP // 128, 128), jnp.float32),
    ),
)


def _tc0_body(x_ref, w1_ref, so_ref, t1_ref):
    t1 = jnp.dot(x_ref[...], w1_ref[...], preferred_element_type=jnp.float32)
    t1_ref[...] = t1 * so_ref[...]


_tc0 = pl.pallas_call(
    _tc0_body,
    out_shape=jax.ShapeDtypeStruct((NP, H), jnp.float32),
)


def _tc1_body(p_ref, si_ref, so_ref, b_ref, w_ref, out_ref):
    agg = (p_ref[0:NP, :] + p_ref[NP:2 * NP, :]) * si_ref[...]
    h = jnp.maximum(agg + b_ref[...], 0.0)
    out_ref[...] = jnp.dot(h, w_ref[...], preferred_element_type=jnp.float32) * so_ref[...]


_tc1 = pl.pallas_call(
    _tc1_body,
    out_shape=jax.ShapeDtypeStruct((NP, H), jnp.float32),
)


def _tc2_body(p_ref, si_ref, so_ref, b_ref, out_ref):
    agg = (p_ref[0:NP, :] + p_ref[NP:2 * NP, :]) * si_ref[...]
    out_ref[...] = jnp.maximum(agg + b_ref[...], 0.0) * so_ref[...]


_tc2 = pl.pallas_call(
    _tc2_body,
    out_shape=jax.ShapeDtypeStruct((NP, H), jnp.float32),
)


def _tc3_body(p_ref, si_ref, b_ref, w_ref, out_ref):
    agg = (p_ref[0:NP, :] + p_ref[NP:2 * NP, :]) * si_ref[...]
    out_ref[...] = jnp.dot(agg, w_ref[...], preferred_element_type=jnp.float32) + b_ref[...]


_tc3 = pl.pallas_call(
    _tc3_body,
    out_shape=jax.ShapeDtypeStruct((NP, D_OUT), jnp.float32),
)


def kernel(x, edge_index, W1, b1, W2, b2, W3, b3):
    src = edge_index[0].astype(jnp.int32).reshape(NW, EPW_REAL)
    dst = edge_index[1].astype(jnp.int32).reshape(NW, EPW_REAL)
    padw = jnp.full((NW, EPW - EPW_REAL), N, jnp.int32)
    src_f = jnp.concatenate([src, padw], axis=1).reshape(NW * EPW)
    dst_f = jnp.concatenate([dst, padw], axis=1).reshape(NW * EPW)
    x_p = jnp.pad(x, ((0, NP - N), (0, 0)))

    deg = _sc_degree(src_f, dst_f).reshape(4 * (NP // 128), 128)
    so_s, si_s = _tcA(deg)
    so = jnp.broadcast_to(so_s.reshape(NP, 1), (NP, H))
    si = jnp.broadcast_to(si_s.reshape(NP, 1), (NP, H))
    t1 = _tc0(x_p, W1, so)
    p1 = _sc_segsum(t1, src_f, dst_f)
    t2 = _tc1(p1, si, so, b1.reshape(1, H), W2)
    p2 = _sc_segsum(t2, src_f, dst_f)
    t3 = _tc2(p2, si, so, b2.reshape(1, H))
    p3 = _sc_segsum(t3, src_f, dst_f)
    out = _tc3(p3, si, b3.reshape(1, D_OUT), W3)
    return out[:N]


# speedup vs baseline: 38.9083x; 1.6031x over previous
import functools

import jax
import jax.numpy as jnp
from jax import lax
from jax.experimental import pallas as pl
from jax.experimental.pallas import tpu as pltpu
from jax.experimental.pallas import tpu_sc as plsc

N = 10000
NP = 10240
E = 320000
H = 16
D_OUT = 100
NC = 2
NS = 16
NW = NC * NS
EPW = E // NW
CH = 2048
CHUNKS = [2048, 2048, 2048, 2048, 1808]
NCHUNK = len(CHUNKS)
RPT = NP // NS
FR = NP // 8
FOLD = 8


def _cparams():
    return pltpu.CompilerParams(use_tc_tiling_on_sc=False)


_mesh_cache = []


def _mesh():
    if not _mesh_cache:
        _mesh_cache.append(plsc.VectorSubcoreMesh(
            core_axis_name="c", subcore_axis_name="s", num_cores=NC, num_subcores=NS
        ))
    return _mesh_cache[0]


def _chunk_off(j):
    return sum(CHUNKS[:j])


def _sc_degree_body(ei_hbm, out_hbm, idx_s, idx_d, ones_v, acc_s, acc_d, sem):
    c = lax.axis_index("c")
    s = lax.axis_index("s")
    wid = s * NC + c
    base = s * RPT

    z = jnp.zeros((H,), jnp.float32)
    for i in range(RPT):
        ones_v[i, :] = z
    pltpu.sync_copy(ones_v.at[pl.ds(0, RPT)], acc_s.at[pl.ds(base, RPT)])
    pltpu.sync_copy(ones_v.at[pl.ds(0, RPT)], acc_d.at[pl.ds(base, RPT)])
    pltpu.sync_copy(ei_hbm.at[0, pl.ds(wid * EPW, EPW)], idx_s)
    pltpu.sync_copy(ei_hbm.at[1, pl.ds(wid * EPW, EPW)], idx_d)
    o = jnp.ones((H,), jnp.float32)
    for i in range(CH):
        ones_v[i, :] = o
    plsc.subcore_barrier()

    descs = []
    for j in range(NCHUNK):
        off, L = _chunk_off(j), CHUNKS[j]
        descs.append(pltpu.async_copy(
            ones_v.at[pl.ds(0, L)], acc_s.at[idx_s.at[pl.ds(off, L)]],
            sem, add=True))
        descs.append(pltpu.async_copy(
            ones_v.at[pl.ds(0, L)], acc_d.at[idx_d.at[pl.ds(off, L)]],
            sem, add=True))
    for d in descs:
        d.wait()
    plsc.subcore_barrier()
    pltpu.sync_copy(acc_s.at[pl.ds(base, RPT)],
                    out_hbm.at[pl.ds(c * (2 * NP) + base, RPT)])
    pltpu.sync_copy(acc_d.at[pl.ds(base, RPT)],
                    out_hbm.at[pl.ds(c * (2 * NP) + NP + base, RPT)])


def _make_sc_degree(interpret=False):
    return functools.partial(
        pl.kernel,
        out_type=jax.ShapeDtypeStruct((NC * 2 * NP, H), jnp.float32),
        mesh=_mesh(),
        scratch_types=[
            pltpu.VMEM((EPW,), jnp.int32),
            pltpu.VMEM((EPW,), jnp.int32),
            pltpu.VMEM((CH, H), jnp.float32),
            pltpu.VMEM_SHARED((NP, H), jnp.float32),
            pltpu.VMEM_SHARED((NP, H), jnp.float32),
            pltpu.SemaphoreType.DMA,
        ],
        compiler_params=_cparams(),
        interpret=interpret,
    )(_sc_degree_body)


def _sc_segsum_body(t_hbm, ei_hbm, out_hbm,
                    idx_s, idx_d, rows0, rows1, t_sp, acc, sem_g, sem_s):
    c = lax.axis_index("c")
    s = lax.axis_index("s")
    wid = s * NC + c
    base = s * RPT
    rows = (rows0, rows1)

    z = jnp.zeros((H,), jnp.float32)
    for i in range(RPT):
        rows0[i, :] = z
    pltpu.sync_copy(rows0.at[pl.ds(0, RPT)], acc.at[pl.ds(base, RPT)])
    pltpu.sync_copy(t_hbm.at[pl.ds(base, RPT)], t_sp.at[pl.ds(base, RPT)])
    pltpu.sync_copy(ei_hbm.at[0, pl.ds(wid * EPW, EPW)], idx_s)
    pltpu.sync_copy(ei_hbm.at[1, pl.ds(wid * EPW, EPW)], idx_d)
    plsc.subcore_barrier()

    def gather(j, buf):
        off, L = _chunk_off(j), CHUNKS[j]
        return pltpu.async_copy(
            t_sp.at[idx_s.at[pl.ds(off, L)]], buf.at[pl.ds(0, L)], sem_g)

    def scatter(j, buf):
        off, L = _chunk_off(j), CHUNKS[j]
        return pltpu.async_copy(
            buf.at[pl.ds(0, L)], acc.at[idx_d.at[pl.ds(off, L)]],
            sem_s, add=True)

    g = [None] * NCHUNK
    sc = [None] * NCHUNK
    g[0] = gather(0, rows[0])
    for j in range(NCHUNK):
        g[j].wait()
        if j + 1 < NCHUNK:
            if j >= 1:
                sc[j - 1].wait()
            g[j + 1] = gather(j + 1, rows[(j + 1) % 2])
        sc[j] = scatter(j, rows[j % 2])
    sc[NCHUNK - 2].wait()
    sc[NCHUNK - 1].wait()
    plsc.subcore_barrier()
    pltpu.sync_copy(acc.at[pl.ds(base, RPT)],
                    out_hbm.at[pl.ds(c * NP + base, RPT)])


def _make_sc_segsum(interpret=False):
    return functools.partial(
        pl.kernel,
        out_type=jax.ShapeDtypeStruct((NC * NP, H), jnp.float32),
        mesh=_mesh(),
        scratch_types=[
            pltpu.VMEM((EPW,), jnp.int32),
            pltpu.VMEM((EPW,), jnp.int32),
            pltpu.VMEM((CH, H), jnp.float32),
            pltpu.VMEM((CH, H), jnp.float32),
            pltpu.VMEM_SHARED((NP, H), jnp.float32),
            pltpu.VMEM_SHARED((NP, H), jnp.float32),
            pltpu.SemaphoreType.DMA,
            pltpu.SemaphoreType.DMA,
        ],
        compiler_params=_cparams(),
        interpret=interpret,
    )(_sc_segsum_body)


_sc_cache = {}


def _sc_degree(*args):
    if "deg" not in _sc_cache:
        _sc_cache["deg"] = _make_sc_degree()
    return _sc_cache["deg"](*args)


def _sc_segsum(*args):
    if "seg" not in _sc_cache:
        _sc_cache["seg"] = _make_sc_segsum()
    return _sc_cache["seg"](*args)


def _block_diag(w, di, dj, reps):
    row = jnp.concatenate([w] * reps, axis=1)
    full = jnp.concatenate([row] * reps, axis=0)
    bi = lax.broadcasted_iota(jnp.int32, (di * reps, dj * reps), 0) // di
    bj = lax.broadcasted_iota(jnp.int32, (di * reps, dj * reps), 1) // dj
    return jnp.where(bi == bj, full, 0.0)


def _valid_mask():
    r = lax.broadcasted_iota(jnp.int32, (FR, 128), 0)
    l = lax.broadcasted_iota(jnp.int32, (FR, 128), 1)
    return (r * FOLD + l // H) < N


def _tc_mm1_body(x8_ref, w1_ref, out_ref):
    w1r = _block_diag(w1_ref[...], 128, H, FOLD)
    out_ref[...] = jnp.dot(x8_ref[...], w1r, preferred_element_type=jnp.float32)


_tc_mm1 = pl.pallas_call(
    _tc_mm1_body,
    out_shape=jax.ShapeDtypeStruct((FR, 128), jnp.float32),
)


def _tcA_body(deg_ref, t1u_ref, so_ref, si_ref, t1_ref):
    d = deg_ref[...]
    deg_out = jnp.maximum(d[0:FR, :] + d[2 * FR:3 * FR, :], 1.0)
    deg_in = jnp.maximum(d[FR:2 * FR, :] + d[3 * FR:4 * FR, :], 1.0)
    valid = _valid_mask()
    so = jnp.where(valid, lax.rsqrt(deg_out), 0.0)
    si = jnp.where(valid, lax.rsqrt(deg_in), 0.0)
    so_ref[...] = so
    si_ref[...] = si
    t1_ref[...] = t1u_ref[...] * so


_tcA = pl.pallas_call(
    _tcA_body,
    out_shape=(
        jax.ShapeDtypeStruct((FR, 128), jnp.float32),
        jax.ShapeDtypeStruct((FR, 128), jnp.float32),
        jax.ShapeDtypeStruct((FR, 128), jnp.float32),
    ),
)


def _tc1_body(p_ref, si_ref, so_ref, b_ref, w_ref, out_ref):
    agg = (p_ref[0:FR, :] + p_ref[FR:2 * FR, :]) * si_ref[...]
    b = jnp.concatenate([b_ref[...]] * FOLD, axis=1)
    h = jnp.maximum(agg + b, 0.0)
    w2r = _block_diag(w_ref[...], H, H, FOLD)
    out_ref[...] = jnp.dot(h, w2r, preferred_element_type=jnp.float32) * so_ref[...]


_tc1 = pl.pallas_call(
    _tc1_body,
    out_shape=jax.ShapeDtypeStruct((FR, 128), jnp.float32),
)


def _tc2_body(p_ref, si_ref, so_ref, b_ref, out_ref):
    agg = (p_ref[0:FR, :] + p_ref[FR:2 * FR, :]) * si_ref[...]
    b = jnp.concatenate([b_ref[...]] * FOLD, axis=1)
    out_ref[...] = jnp.maximum(agg + b, 0.0) * so_ref[...]


_tc2 = pl.pallas_call(
    _tc2_body,
    out_shape=jax.ShapeDtypeStruct((FR, 128), jnp.float32),
)


def _tc3_body(p_ref, si_ref, b_ref, w_ref, out_ref):
    agg = (p_ref[0:FR, :] + p_ref[FR:2 * FR, :]) * si_ref[...]
    w3r = _block_diag(w_ref[...], H, D_OUT, FOLD)
    b = jnp.concatenate([b_ref[...]] * FOLD, axis=1)
    out_ref[...] = jnp.dot(agg, w3r, preferred_element_type=jnp.float32) + b


_tc3 = pl.pallas_call(
    _tc3_body,
    out_shape=jax.ShapeDtypeStruct((FR, FOLD * D_OUT), jnp.float32),
)


def kernel(x, edge_index, W1, b1, W2, b2, W3, b3):
    ei = edge_index.astype(jnp.int32)
    x8 = jnp.pad(x, ((0, NP - N), (0, 0))).reshape(FR, FOLD * 128)

    deg = _sc_degree(ei)
    deg_f = deg.reshape(4 * FR, 128)
    t1u = _tc_mm1(x8, W1)
    so, si, t1f = _tcA(deg_f, t1u)
    p1 = _sc_segsum(t1f.reshape(NP, H), ei).reshape(2 * FR, 128)
    t2f = _tc1(p1, si, so, b1.reshape(1, H), W2)
    p2 = _sc_segsum(t2f.reshape(NP, H), ei).reshape(2 * FR, 128)
    t3f = _tc2(p2, si, so, b2.reshape(1, H))
    p3 = _sc_segsum(t3f.reshape(NP, H), ei).reshape(2 * FR, 128)
    outf = _tc3(p3, si, b3.reshape(1, D_OUT), W3)
    return outf.reshape(NP, D_OUT)[:N]
